# Initial kernel scaffold; baseline (speedup 1.0000x reference)
import jax
import jax.numpy as jnp
from jax.experimental import pallas as pl


def kernel(x, edge_index, graph_batch, Wn1, Wr1, b1, Wn2, Wr2, b2, Wn3, Wr3, b3, g1, bt1, g2, bt2, g3, bt3, Wro1, bro1, Wro2, bro2):
    raise NotImplementedError("write your pallas kernel here")


# speedup vs baseline: 4.8449x; 4.8449x over previous
import functools

import jax
import jax.numpy as jnp
from jax import lax
from jax.experimental import pallas as pl
from jax.experimental.pallas import tpu as pltpu
from jax.experimental.pallas import tpu_sc as plsc

N = 10000
E = 320000
H = 128
G = 64
OUT = 256
EPS = 1e-5

NC = 2
NS = 16
NW = NC * NS
L = 16

EPT = E // NW
ECH = 128
NFULL = EPT // ECH
ETAIL = EPT - NFULL * ECH

RPT = 624
RTAIL = N - NS * RPT

PN = 312
PTAIL = N - PN * NW

_F32 = jnp.float32
_I32 = jnp.int32

_mesh = plsc.VectorSubcoreMesh(
    core_axis_name="c", subcore_axis_name="s", num_cores=NC, num_subcores=NS)

_HIGH = lax.Precision.HIGHEST


def _slab_zero(zeros_hbm, acc, s):
    r0 = s * RPT
    pltpu.sync_copy(zeros_hbm.at[pl.ds(r0, RPT)], acc.at[pl.ds(r0, RPT)])

    @pl.when(s == NS - 1)
    def _tail():
        t0 = NS * RPT
        pltpu.sync_copy(zeros_hbm.at[pl.ds(t0, RTAIL)],
                        acc.at[pl.ds(t0, RTAIL)])


def _slab_out(acc, out_hbm, c, s):
    r0 = s * RPT
    pltpu.sync_copy(acc.at[pl.ds(r0, RPT)], out_hbm.at[pl.ds(c * N + r0, RPT)])

    @pl.when(s == NS - 1)
    def _tail():
        t0 = NS * RPT
        pltpu.sync_copy(acc.at[pl.ds(t0, RTAIL)],
                        out_hbm.at[pl.ds(c * N + t0, RTAIL)])


@functools.partial(
    pl.kernel,
    out_type=jax.ShapeDtypeStruct((NC * N, H), _F32),
    mesh=_mesh,
    scratch_types=[
        pltpu.VMEM((ECH,), _I32),
        pltpu.VMEM((ETAIL,), _I32),
        pltpu.VMEM((ECH, H), _F32),
        pltpu.VMEM((ETAIL, H), _F32),
        pltpu.VMEM_SHARED((N, H), _F32),
    ],
)
def _deg_sc(dst_hbm, zeros_hbm, ones_hbm, out_hbm, idx_d, idx_d2, ones_v,
            ones_v2, acc):
    c = lax.axis_index("c")
    s = lax.axis_index("s")
    wid = c * NS + s
    _slab_zero(zeros_hbm, acc, s)
    pltpu.sync_copy(ones_hbm.at[pl.ds(0, ECH)], ones_v)
    pltpu.sync_copy(ones_hbm.at[pl.ds(0, ETAIL)], ones_v2)
    plsc.subcore_barrier()
    e0 = wid * EPT

    @pl.loop(0, NFULL)
    def _chunk(k):
        base = e0 + k * ECH
        pltpu.sync_copy(dst_hbm.at[pl.ds(base, ECH)], idx_d)
        pltpu.sync_copy(ones_v, acc.at[idx_d], add=True)

    base = e0 + NFULL * ECH
    pltpu.sync_copy(dst_hbm.at[pl.ds(base, ETAIL)], idx_d2)
    pltpu.sync_copy(ones_v2, acc.at[idx_d2], add=True)
    plsc.subcore_barrier()
    _slab_out(acc, out_hbm, c, s)


@functools.partial(
    pl.kernel,
    out_type=jax.ShapeDtypeStruct((NC * N, H), _F32),
    mesh=_mesh,
    scratch_types=[
        pltpu.VMEM((ECH,), _I32),
        pltpu.VMEM((ECH,), _I32),
        pltpu.VMEM((ECH, H), _F32),
        pltpu.VMEM((ETAIL,), _I32),
        pltpu.VMEM((ETAIL,), _I32),
        pltpu.VMEM((ETAIL, H), _F32),
        pltpu.VMEM_SHARED((N, H), _F32),
        pltpu.SemaphoreType.DMA,
    ],
)
def _edge_sc(h_hbm, src_hbm, dst_hbm, zeros_hbm, out_hbm, idx_s, idx_d, rows,
             idx_s2, idx_d2, rows2, acc, sem):
    c = lax.axis_index("c")
    s = lax.axis_index("s")
    wid = c * NS + s
    _slab_zero(zeros_hbm, acc, s)
    plsc.subcore_barrier()
    e0 = wid * EPT

    @pl.loop(0, NFULL)
    def _chunk(k):
        base = e0 + k * ECH
        pltpu.sync_copy(src_hbm.at[pl.ds(base, ECH)], idx_s)
        pltpu.sync_copy(dst_hbm.at[pl.ds(base, ECH)], idx_d)
        pltpu.async_copy(h_hbm.at[idx_s], rows, sem).wait()
        pltpu.sync_copy(rows, acc.at[idx_d], add=True)

    base = e0 + NFULL * ECH
    pltpu.sync_copy(src_hbm.at[pl.ds(base, ETAIL)], idx_s2)
    pltpu.sync_copy(dst_hbm.at[pl.ds(base, ETAIL)], idx_d2)
    pltpu.async_copy(h_hbm.at[idx_s2], rows2, sem).wait()
    pltpu.sync_copy(rows2, acc.at[idx_d2], add=True)
    plsc.subcore_barrier()
    _slab_out(acc, out_hbm, c, s)


NCH = N // ECH
NTAIL = N - NCH * ECH
CPT = (NCH + NW - 1) // NW


@functools.partial(
    pl.kernel,
    out_type=(
        jax.ShapeDtypeStruct((NC * G, H), _F32),
        jax.ShapeDtypeStruct((NC * G, H), _F32),
    ),
    mesh=_mesh,
    scratch_types=[
        pltpu.VMEM((ECH,), _I32),
        pltpu.VMEM((ECH, H), _F32),
        pltpu.VMEM((ECH, H), _F32),
        pltpu.VMEM((NTAIL,), _I32),
        pltpu.VMEM((NTAIL, H), _F32),
        pltpu.VMEM_SHARED((G, H), _F32),
        pltpu.VMEM_SHARED((G, H), _F32),
        pltpu.SemaphoreType.DMA,
    ],
)
def _pool_sc(h_hbm, gb_hbm, zerosh_hbm, ones_hbm, sum_hbm,
             cnt_hbm, gbv, rows, ones_v, gbv2, rows2, accs, accc, sem):
    c = lax.axis_index("c")
    s = lax.axis_index("s")
    wid = c * NS + s

    @pl.when(s < G // 8)
    def _zero():
        r0 = s * 8
        pltpu.sync_copy(zerosh_hbm.at[pl.ds(r0, 8)], accs.at[pl.ds(r0, 8)])
        pltpu.sync_copy(zerosh_hbm.at[pl.ds(G + r0, 8)],
                        accc.at[pl.ds(r0, 8)])

    pltpu.sync_copy(ones_hbm, ones_v)
    plsc.subcore_barrier()

    @pl.loop(0, CPT)
    def _chunk(k):
        cid = wid + k * NW

        @pl.when(cid < NCH)
        def _go():
            base = cid * ECH
            pltpu.sync_copy(gb_hbm.at[pl.ds(base, ECH)], gbv)
            pltpu.sync_copy(h_hbm.at[pl.ds(base, ECH)], rows)
            pltpu.sync_copy(rows, accs.at[gbv], add=True)
            pltpu.sync_copy(ones_v, accc.at[gbv], add=True)

    @pl.when(wid == NW - 1)
    def _tail():
        tb = NCH * ECH
        pltpu.sync_copy(gb_hbm.at[pl.ds(tb, NTAIL)], gbv2)
        pltpu.sync_copy(h_hbm.at[pl.ds(tb, NTAIL)], rows2)
        pltpu.sync_copy(rows2, accs.at[gbv2], add=True)
        pltpu.sync_copy(ones_v.at[pl.ds(0, NTAIL)], accc.at[gbv2], add=True)

    plsc.subcore_barrier()

    @pl.when(s < G // 8)
    def _out():
        r0 = s * 8
        pltpu.sync_copy(accs.at[pl.ds(r0, 8)],
                        sum_hbm.at[pl.ds(c * G + r0, 8)])
        pltpu.sync_copy(accc.at[pl.ds(r0, 8)],
                        cnt_hbm.at[pl.ds(c * G + r0, 8)])


def _max_body(gb_ref, h_ref, o_ref, acc):
    i = pl.program_id(0)

    @pl.when(i == 0)
    def _init():
        acc[...] = jnp.full((G, H), -jnp.inf, _F32)

    gbv = gb_ref[...]
    hb = h_ref[...]
    for g in range(G):
        m = jnp.where(gbv == g, hb, -jnp.inf)
        acc[g:g + 1, :] = jnp.maximum(acc[g:g + 1, :],
                                      jnp.max(m, axis=0, keepdims=True))

    o_ref[...] = acc[...]


def _max_tc(gb2d, h):
    return pl.pallas_call(
        _max_body,
        grid=(N // _RB,),
        in_specs=[
            pl.BlockSpec((_RB, 1), lambda i: (i, 0)),
            pl.BlockSpec((_RB, H), lambda i: (i, 0)),
        ],
        out_specs=pl.BlockSpec((G, H), lambda i: (0, 0)),
        out_shape=jax.ShapeDtypeStruct((G, H), _F32),
        scratch_shapes=[pltpu.VMEM((G, H), _F32)],
    )(gb2d, h)


_RB = 400


def _layer_body(aggp_ref, degp_ref, h_ref, wn_ref, wr_ref, b_ref, g_ref,
                bt_ref, o_ref):
    agg = aggp_ref[0] + aggp_ref[1]
    deg = degp_ref[0, :, 0:1] + degp_ref[1, :, 0:1]
    mean = agg * (1.0 / jnp.maximum(deg, 1.0))
    z = (jnp.dot(mean, wn_ref[...], preferred_element_type=_F32,
                 precision=_HIGH)
         + jnp.dot(h_ref[...], wr_ref[...], preferred_element_type=_F32,
                   precision=_HIGH)
         + b_ref[...])
    mu = jnp.mean(z, axis=1, keepdims=True)
    var = jnp.mean((z - mu) * (z - mu), axis=1, keepdims=True)
    zn = (z - mu) * lax.rsqrt(var + EPS) * g_ref[...] + bt_ref[...]
    o_ref[...] = jnp.maximum(zn, 0.0)


def _layer_tc(aggp, degp, h, wn, wr, b, g, bt):
    grid = (N // _RB,)
    return pl.pallas_call(
        _layer_body,
        grid=grid,
        in_specs=[
            pl.BlockSpec((NC, _RB, H), lambda i: (0, i, 0)),
            pl.BlockSpec((NC, _RB, H), lambda i: (0, i, 0)),
            pl.BlockSpec((_RB, H), lambda i: (i, 0)),
            pl.BlockSpec((H, H), lambda i: (0, 0)),
            pl.BlockSpec((H, H), lambda i: (0, 0)),
            pl.BlockSpec((1, H), lambda i: (0, 0)),
            pl.BlockSpec((1, H), lambda i: (0, 0)),
            pl.BlockSpec((1, H), lambda i: (0, 0)),
        ],
        out_specs=pl.BlockSpec((_RB, H), lambda i: (i, 0)),
        out_shape=jax.ShapeDtypeStruct((N, H), _F32),
    )(aggp, degp, h, wn, wr, b.reshape(1, H), g.reshape(1, H),
      bt.reshape(1, H))


def _readout_body(sump_ref, maxv_ref, cntp_ref, w1_ref, b1_ref, w2_ref,
                  b2_ref, o_ref):
    sums = sump_ref[0] + sump_ref[1]
    maxs = maxv_ref[...]
    cnt = cntp_ref[0, :, 0:1] + cntp_ref[1, :, 0:1]
    mean = sums / jnp.maximum(cnt, 1.0)
    maxs = jnp.where(jnp.isfinite(maxs), maxs, 0.0)
    pooled = jnp.concatenate([mean, maxs], axis=1)
    z = jnp.maximum(
        jnp.dot(pooled, w1_ref[...], preferred_element_type=_F32,
                precision=_HIGH) + b1_ref[...], 0.0)
    emb = jnp.dot(z, w2_ref[...], preferred_element_type=_F32,
                  precision=_HIGH) + b2_ref[...]
    nrm = jnp.sqrt(jnp.sum(emb * emb, axis=1, keepdims=True))
    o_ref[...] = emb / jnp.maximum(nrm, 1e-12)


def _readout_tc(sump, maxv, cntp, w1, b1, w2, b2):
    return pl.pallas_call(
        _readout_body,
        out_shape=jax.ShapeDtypeStruct((G, OUT), _F32),
    )(sump, maxv, cntp, w1, b1.reshape(1, H), w2, b2.reshape(1, OUT))


def kernel(x, edge_index, graph_batch, Wn1, Wr1, b1, Wn2, Wr2, b2, Wn3, Wr3,
           b3, g1, bt1, g2, bt2, g3, bt3, Wro1, bro1, Wro2, bro2):
    src = edge_index[0]
    dst = edge_index[1]
    zeros_nh = jnp.zeros((N, H), _F32)
    ones_eh = jnp.ones((ECH, H), _F32)

    degp = _deg_sc(dst, zeros_nh, ones_eh).reshape(NC, N, H)

    h = x
    for wn, wr, b, g, bt in ((Wn1, Wr1, b1, g1, bt1),
                             (Wn2, Wr2, b2, g2, bt2),
                             (Wn3, Wr3, b3, g3, bt3)):
        aggp = _edge_sc(h, src, dst, zeros_nh).reshape(NC, N, H)
        h = _layer_tc(aggp, degp, h, wn, wr, b, g, bt)

    sump, cntp = _pool_sc(h, graph_batch, zeros_nh, ones_eh)
    maxv = _max_tc(graph_batch.reshape(N, 1), h)
    return _readout_tc(sump.reshape(NC, G, H), maxv,
                       cntp.reshape(NC, G, H), Wro1, bro1, Wro2, bro2)


# speedup vs baseline: 6.5974x; 1.3617x over previous
import functools

import jax
import jax.numpy as jnp
from jax import lax
from jax.experimental import pallas as pl
from jax.experimental.pallas import tpu as pltpu
from jax.experimental.pallas import tpu_sc as plsc

N = 10000
E = 320000
H = 128
G = 64
OUT = 256
EPS = 1e-5

NC = 2
NS = 16
NW = NC * NS
L = 16

EPT = E // NW
ECH = 128
NFULL = EPT // ECH
ETAIL = EPT - NFULL * ECH

RPT = 624
RTAIL = N - NS * RPT

PN = 312
PTAIL = N - PN * NW

_F32 = jnp.float32
_I32 = jnp.int32

_mesh = plsc.VectorSubcoreMesh(
    core_axis_name="c", subcore_axis_name="s", num_cores=NC, num_subcores=NS)

_HIGH = lax.Precision.HIGHEST


def _slab_zero(zeros_hbm, acc, s):
    r0 = s * RPT
    pltpu.sync_copy(zeros_hbm.at[pl.ds(r0, RPT)], acc.at[pl.ds(r0, RPT)])

    @pl.when(s == NS - 1)
    def _tail():
        t0 = NS * RPT
        pltpu.sync_copy(zeros_hbm.at[pl.ds(t0, RTAIL)],
                        acc.at[pl.ds(t0, RTAIL)])


def _slab_out(acc, out_hbm, c, s):
    r0 = s * RPT
    pltpu.sync_copy(acc.at[pl.ds(r0, RPT)], out_hbm.at[pl.ds(c * N + r0, RPT)])

    @pl.when(s == NS - 1)
    def _tail():
        t0 = NS * RPT
        pltpu.sync_copy(acc.at[pl.ds(t0, RTAIL)],
                        out_hbm.at[pl.ds(c * N + t0, RTAIL)])


@functools.partial(
    pl.kernel,
    out_type=jax.ShapeDtypeStruct((NC * N, H), _F32),
    mesh=_mesh,
    scratch_types=[
        pltpu.VMEM((ECH,), _I32),
        pltpu.VMEM((ETAIL,), _I32),
        pltpu.VMEM((ECH, H), _F32),
        pltpu.VMEM((ETAIL, H), _F32),
        pltpu.VMEM_SHARED((N, H), _F32),
    ],
)
def _deg_sc(dst_hbm, zeros_hbm, ones_hbm, out_hbm, idx_d, idx_d2, ones_v,
            ones_v2, acc):
    c = lax.axis_index("c")
    s = lax.axis_index("s")
    wid = c * NS + s
    _slab_zero(zeros_hbm, acc, s)
    pltpu.sync_copy(ones_hbm.at[pl.ds(0, ECH)], ones_v)
    pltpu.sync_copy(ones_hbm.at[pl.ds(0, ETAIL)], ones_v2)
    plsc.subcore_barrier()
    e0 = wid * EPT

    @pl.loop(0, NFULL)
    def _chunk(k):
        base = e0 + k * ECH
        pltpu.sync_copy(dst_hbm.at[pl.ds(base, ECH)], idx_d)
        pltpu.sync_copy(ones_v, acc.at[idx_d], add=True)

    base = e0 + NFULL * ECH
    pltpu.sync_copy(dst_hbm.at[pl.ds(base, ETAIL)], idx_d2)
    pltpu.sync_copy(ones_v2, acc.at[idx_d2], add=True)
    plsc.subcore_barrier()
    _slab_out(acc, out_hbm, c, s)


PAIRS = NFULL // 2


@functools.partial(
    pl.kernel,
    out_type=jax.ShapeDtypeStruct((NC * N, H), _F32),
    mesh=_mesh,
    scratch_types=[
        pltpu.VMEM((ECH,), _I32),
        pltpu.VMEM((ECH,), _I32),
        pltpu.VMEM((ECH,), _I32),
        pltpu.VMEM((ECH,), _I32),
        pltpu.VMEM((ECH, H), _F32),
        pltpu.VMEM((ECH, H), _F32),
        pltpu.VMEM((ETAIL,), _I32),
        pltpu.VMEM((ETAIL,), _I32),
        pltpu.VMEM((ETAIL, H), _F32),
        pltpu.VMEM_SHARED((N, H), _F32),
        pltpu.SemaphoreType.DMA,
        pltpu.SemaphoreType.DMA,
        pltpu.SemaphoreType.DMA,
        pltpu.SemaphoreType.DMA,
    ],
)
def _edge_sc(h_hbm, src_hbm, dst_hbm, zeros_hbm, out_hbm, idx_s0, idx_d0,
             idx_s1, idx_d1, rows0, rows1, idx_s2, idx_d2, rows2, acc,
             gsem0, gsem1, ssem0, ssem1):
    c = lax.axis_index("c")
    s = lax.axis_index("s")
    wid = c * NS + s
    _slab_zero(zeros_hbm, acc, s)
    plsc.subcore_barrier()
    e0 = wid * EPT

    @pl.loop(0, PAIRS)
    def _pair(p):
        b0 = e0 + (2 * p) * ECH
        b1 = b0 + ECH

        @pl.when(p > 0)
        def _w0():
            pltpu.make_async_copy(rows0, acc.at[idx_d0], ssem0).wait()

        pltpu.sync_copy(src_hbm.at[pl.ds(b0, ECH)], idx_s0)
        pltpu.sync_copy(dst_hbm.at[pl.ds(b0, ECH)], idx_d0)
        pltpu.async_copy(h_hbm.at[idx_s0], rows0, gsem0)

        @pl.when(p > 0)
        def _w1():
            pltpu.make_async_copy(rows1, acc.at[idx_d1], ssem1).wait()

        pltpu.sync_copy(src_hbm.at[pl.ds(b1, ECH)], idx_s1)
        pltpu.sync_copy(dst_hbm.at[pl.ds(b1, ECH)], idx_d1)
        pltpu.async_copy(h_hbm.at[idx_s1], rows1, gsem1)

        pltpu.make_async_copy(h_hbm.at[idx_s0], rows0, gsem0).wait()
        pltpu.async_copy(rows0, acc.at[idx_d0], ssem0, add=True)
        pltpu.make_async_copy(h_hbm.at[idx_s1], rows1, gsem1).wait()
        pltpu.async_copy(rows1, acc.at[idx_d1], ssem1, add=True)

    pltpu.make_async_copy(rows0, acc.at[idx_d0], ssem0).wait()
    pltpu.make_async_copy(rows1, acc.at[idx_d1], ssem1).wait()
    base = e0 + NFULL * ECH
    pltpu.sync_copy(src_hbm.at[pl.ds(base, ETAIL)], idx_s2)
    pltpu.sync_copy(dst_hbm.at[pl.ds(base, ETAIL)], idx_d2)
    pltpu.async_copy(h_hbm.at[idx_s2], rows2, gsem0).wait()
    pltpu.sync_copy(rows2, acc.at[idx_d2], add=True)
    plsc.subcore_barrier()
    _slab_out(acc, out_hbm, c, s)


NCH = N // ECH
NTAIL = N - NCH * ECH
CPT = (NCH + NW - 1) // NW


@functools.partial(
    pl.kernel,
    out_type=(
        jax.ShapeDtypeStruct((NC * G, H), _F32),
        jax.ShapeDtypeStruct((NC * G, H), _F32),
    ),
    mesh=_mesh,
    scratch_types=[
        pltpu.VMEM((ECH,), _I32),
        pltpu.VMEM((ECH, H), _F32),
        pltpu.VMEM((ECH, H), _F32),
        pltpu.VMEM((NTAIL,), _I32),
        pltpu.VMEM((NTAIL, H), _F32),
        pltpu.VMEM_SHARED((G, H), _F32),
        pltpu.VMEM_SHARED((G, H), _F32),
        pltpu.SemaphoreType.DMA,
    ],
)
def _pool_sc(h_hbm, gb_hbm, zerosh_hbm, ones_hbm, sum_hbm,
             cnt_hbm, gbv, rows, ones_v, gbv2, rows2, accs, accc, sem):
    c = lax.axis_index("c")
    s = lax.axis_index("s")
    wid = c * NS + s

    @pl.when(s < G // 8)
    def _zero():
        r0 = s * 8
        pltpu.sync_copy(zerosh_hbm.at[pl.ds(r0, 8)], accs.at[pl.ds(r0, 8)])
        pltpu.sync_copy(zerosh_hbm.at[pl.ds(G + r0, 8)],
                        accc.at[pl.ds(r0, 8)])

    pltpu.sync_copy(ones_hbm, ones_v)
    plsc.subcore_barrier()

    @pl.loop(0, CPT)
    def _chunk(k):
        cid = wid + k * NW

        @pl.when(cid < NCH)
        def _go():
            base = cid * ECH
            pltpu.sync_copy(gb_hbm.at[pl.ds(base, ECH)], gbv)
            pltpu.sync_copy(h_hbm.at[pl.ds(base, ECH)], rows)
            pltpu.sync_copy(rows, accs.at[gbv], add=True)
            pltpu.sync_copy(ones_v, accc.at[gbv], add=True)

    @pl.when(wid == NW - 1)
    def _tail():
        tb = NCH * ECH
        pltpu.sync_copy(gb_hbm.at[pl.ds(tb, NTAIL)], gbv2)
        pltpu.sync_copy(h_hbm.at[pl.ds(tb, NTAIL)], rows2)
        pltpu.sync_copy(rows2, accs.at[gbv2], add=True)
        pltpu.sync_copy(ones_v.at[pl.ds(0, NTAIL)], accc.at[gbv2], add=True)

    plsc.subcore_barrier()

    @pl.when(s < G // 8)
    def _out():
        r0 = s * 8
        pltpu.sync_copy(accs.at[pl.ds(r0, 8)],
                        sum_hbm.at[pl.ds(c * G + r0, 8)])
        pltpu.sync_copy(accc.at[pl.ds(r0, 8)],
                        cnt_hbm.at[pl.ds(c * G + r0, 8)])


def _max_body(gb_ref, h_ref, o_ref, acc):
    i = pl.program_id(0)

    @pl.when(i == 0)
    def _init():
        acc[...] = jnp.full((G, H), -jnp.inf, _F32)

    gbv = gb_ref[...]
    hb = h_ref[...]
    for g in range(G):
        m = jnp.where(gbv == g, hb, -jnp.inf)
        acc[g:g + 1, :] = jnp.maximum(acc[g:g + 1, :],
                                      jnp.max(m, axis=0, keepdims=True))

    o_ref[...] = acc[...]


def _max_tc(gb2d, h):
    return pl.pallas_call(
        _max_body,
        grid=(N // _RB,),
        in_specs=[
            pl.BlockSpec((_RB, 1), lambda i: (i, 0)),
            pl.BlockSpec((_RB, H), lambda i: (i, 0)),
        ],
        out_specs=pl.BlockSpec((G, H), lambda i: (0, 0)),
        out_shape=jax.ShapeDtypeStruct((G, H), _F32),
        scratch_shapes=[pltpu.VMEM((G, H), _F32)],
    )(gb2d, h)


_RB = 400


def _layer_body(aggp_ref, degp_ref, h_ref, wn_ref, wr_ref, b_ref, g_ref,
                bt_ref, o_ref):
    agg = aggp_ref[0] + aggp_ref[1]
    deg = degp_ref[0, :, 0:1] + degp_ref[1, :, 0:1]
    mean = agg * (1.0 / jnp.maximum(deg, 1.0))
    z = (jnp.dot(mean, wn_ref[...], preferred_element_type=_F32,
                 precision=_HIGH)
         + jnp.dot(h_ref[...], wr_ref[...], preferred_element_type=_F32,
                   precision=_HIGH)
         + b_ref[...])
    mu = jnp.mean(z, axis=1, keepdims=True)
    var = jnp.mean((z - mu) * (z - mu), axis=1, keepdims=True)
    zn = (z - mu) * lax.rsqrt(var + EPS) * g_ref[...] + bt_ref[...]
    o_ref[...] = jnp.maximum(zn, 0.0)


def _layer_tc(aggp, degp, h, wn, wr, b, g, bt):
    grid = (N // _RB,)
    return pl.pallas_call(
        _layer_body,
        grid=grid,
        in_specs=[
            pl.BlockSpec((NC, _RB, H), lambda i: (0, i, 0)),
            pl.BlockSpec((NC, _RB, H), lambda i: (0, i, 0)),
            pl.BlockSpec((_RB, H), lambda i: (i, 0)),
            pl.BlockSpec((H, H), lambda i: (0, 0)),
            pl.BlockSpec((H, H), lambda i: (0, 0)),
            pl.BlockSpec((1, H), lambda i: (0, 0)),
            pl.BlockSpec((1, H), lambda i: (0, 0)),
            pl.BlockSpec((1, H), lambda i: (0, 0)),
        ],
        out_specs=pl.BlockSpec((_RB, H), lambda i: (i, 0)),
        out_shape=jax.ShapeDtypeStruct((N, H), _F32),
    )(aggp, degp, h, wn, wr, b.reshape(1, H), g.reshape(1, H),
      bt.reshape(1, H))


def _readout_body(sump_ref, maxv_ref, cntp_ref, w1_ref, b1_ref, w2_ref,
                  b2_ref, o_ref):
    sums = sump_ref[0] + sump_ref[1]
    maxs = maxv_ref[...]
    cnt = cntp_ref[0, :, 0:1] + cntp_ref[1, :, 0:1]
    mean = sums / jnp.maximum(cnt, 1.0)
    maxs = jnp.where(jnp.isfinite(maxs), maxs, 0.0)
    pooled = jnp.concatenate([mean, maxs], axis=1)
    z = jnp.maximum(
        jnp.dot(pooled, w1_ref[...], preferred_element_type=_F32,
                precision=_HIGH) + b1_ref[...], 0.0)
    emb = jnp.dot(z, w2_ref[...], preferred_element_type=_F32,
                  precision=_HIGH) + b2_ref[...]
    nrm = jnp.sqrt(jnp.sum(emb * emb, axis=1, keepdims=True))
    o_ref[...] = emb / jnp.maximum(nrm, 1e-12)


def _readout_tc(sump, maxv, cntp, w1, b1, w2, b2):
    return pl.pallas_call(
        _readout_body,
        out_shape=jax.ShapeDtypeStruct((G, OUT), _F32),
    )(sump, maxv, cntp, w1, b1.reshape(1, H), w2, b2.reshape(1, OUT))


def kernel(x, edge_index, graph_batch, Wn1, Wr1, b1, Wn2, Wr2, b2, Wn3, Wr3,
           b3, g1, bt1, g2, bt2, g3, bt3, Wro1, bro1, Wro2, bro2):
    src = edge_index[0]
    dst = edge_index[1]
    zeros_nh = jnp.zeros((N, H), _F32)
    ones_eh = jnp.ones((ECH, H), _F32)

    degp = _deg_sc(dst, zeros_nh, ones_eh).reshape(NC, N, H)

    h = x
    for wn, wr, b, g, bt in ((Wn1, Wr1, b1, g1, bt1),
                             (Wn2, Wr2, b2, g2, bt2),
                             (Wn3, Wr3, b3, g3, bt3)):
        aggp = _edge_sc(h, src, dst, zeros_nh).reshape(NC, N, H)
        h = _layer_tc(aggp, degp, h, wn, wr, b, g, bt)

    sump, cntp = _pool_sc(h, graph_batch, zeros_nh, ones_eh)
    maxv = _max_tc(graph_batch.reshape(N, 1), h)
    return _readout_tc(sump.reshape(NC, G, H), maxv,
                       cntp.reshape(NC, G, H), Wro1, bro1, Wro2, bro2)
